# baseline (speedup 1.0000x reference)
import functools

import jax
import jax.numpy as jnp
from jax import lax
from jax.experimental import pallas as pl
from jax.experimental.pallas import tpu as pltpu
from jax.experimental.pallas import tpu_sc as plsc

_NW = 32
_TILE = 1024
_CHUNK_TILES = 16
_CHUNK = _CHUNK_TILES * _TILE


def _swap_bits(q, a, b):
    x = ((q >> a) ^ (q >> b)) & 1
    return q ^ ((x << a) | (x << b))


def _chunk_perm():
    p = jnp.arange(_CHUNK, dtype=jnp.int32)
    q = p & (_TILE - 1)
    for a, b in ((9, 5), (8, 6), (4, 0), (3, 1)):
        q = _swap_bits(q, a, b)
    return (p & ~(_TILE - 1)) | q


def _sc_body(x_hbm, perm_hbm, out_hbm, perm_v, in_v, out_v):
    c = lax.axis_index("c")
    s = lax.axis_index("s")
    wid = s * 2 + c
    base = wid * (16384 // _NW) * _TILE
    pltpu.sync_copy(perm_hbm, perm_v)

    def chunk_body(k, carry):
        off = base + k * _CHUNK
        pltpu.sync_copy(x_hbm.at[pl.ds(off, _CHUNK)], in_v)

        @plsc.parallel_loop(0, _CHUNK // 16, unroll=8)
        def grp(i):
            idx = perm_v[pl.ds(i * 16, 16)]
            out_v[pl.ds(i * 16, 16)] = plsc.load_gather(in_v, [idx])
        pltpu.sync_copy(out_v, out_hbm.at[pl.ds(off, _CHUNK)])
        return carry

    lax.fori_loop(0, (16384 // _NW) // _CHUNK_TILES, chunk_body, 0)


def kernel(tensor):
    n, r, c = tensor.shape
    xf = tensor.reshape(n * r * c)
    perm = _chunk_perm()
    mesh = plsc.VectorSubcoreMesh(core_axis_name="c", subcore_axis_name="s")
    kfn = functools.partial(
        pl.kernel,
        out_type=jax.ShapeDtypeStruct((n * r * c,), tensor.dtype),
        mesh=mesh,
        scratch_types=[
            pltpu.VMEM((_CHUNK,), jnp.int32),
            pltpu.VMEM((_CHUNK,), jnp.float32),
            pltpu.VMEM((_CHUNK,), jnp.float32),
        ],
        compiler_params=pltpu.CompilerParams(needs_layout_passes=False),
    )(_sc_body)
    out = kfn(xf, perm)
    return out.reshape(n, r, c)
